# baseline (speedup 1.0000x reference)
import functools

import jax
import jax.numpy as jnp
from jax import lax
from jax.experimental import pallas as pl
from jax.experimental.pallas import tpu as pltpu
from jax.experimental.pallas import tpu_sc as plsc

_B = 16384
_C = 1000
_CP = 1024
_BM = 2048
_NW = 32
_SC_ROWS = 2048
_TC_ROWS = _B - _SC_ROWS
_GRID = _TC_ROWS // _BM
_ROWS = _SC_ROWS // _NW
_CH = 32
_NCH = _ROWS // _CH


def _tc_body(x_ref, lab_ref, out_ref, acc_ref):
    step = pl.program_id(0)

    @pl.when(step == 0)
    def _init():
        acc_ref[...] = jnp.zeros_like(acc_ref)

    x = x_ref[...]
    m = jnp.max(x, axis=1, keepdims=True)
    idx = lax.broadcasted_iota(jnp.int32, (_BM, _C), 1)
    masked = jnp.where(x == m, idx, _C)
    pred = jnp.min(masked, axis=1)

    labels = lab_ref[0, 0, :]
    correct = (pred == labels).astype(jnp.float32)

    cls = lax.broadcasted_iota(jnp.int32, (_BM, _CP), 1)
    onehot = (pred[:, None] == cls).astype(jnp.float32)
    w = jnp.concatenate(
        [jnp.ones((1, _BM), jnp.float32), correct[None, :]], axis=0
    )
    acc_ref[...] += jax.lax.dot(
        w, onehot, preferred_element_type=jnp.float32
    )

    @pl.when(step == _GRID - 1)
    def _fini():
        out_ref[...] = acc_ref[...]


def _sc_body(x_hbm, lab_hbm, out_hbm,
             xa, xb, lab_v, hpp, htp, part_pp, part_tp, sem_a, sem_b):
    cid = lax.axis_index("c")
    sid = lax.axis_index("s")
    w = sid * 2 + cid
    base = w * _ROWS

    pltpu.sync_copy(lab_hbm.at[pl.ds(_TC_ROWS + base, _ROWS)], lab_v)

    z16 = jnp.zeros((16,), jnp.float32)

    def _zero(j, _):
        for u in range(16):
            hpp[pl.ds(j * 256 + u * 16, 16)] = z16
            htp[pl.ds(j * 256 + u * 16, 16)] = z16
        return 0

    lax.fori_loop(0, 16 * _CP // 256, _zero, 0)

    lane = lax.iota(jnp.int32, 16)
    lane_base = lane * _CP
    ones = jnp.ones((16,), jnp.float32)
    neg_inf = jnp.full((16,), -jnp.inf, jnp.float32)
    zi16 = jnp.zeros((16,), jnp.int32)

    def _chunk(x_v, c16):
        for g in range(_CH // 16):
            rows = lane + g * 16

            def _cls(cc, carry):
                m0, m1, m2, m3, p0, p1, p2, p3 = carry
                ms = [m0, m1, m2, m3]
                ps = [p0, p1, p2, p3]
                for u in range(8):
                    c = cc * 8 + u
                    q = u % 4
                    col = jnp.full((16,), 0, jnp.int32) + c
                    v = plsc.load_gather(x_v, [rows, col])
                    take = v > ms[q]
                    ms[q] = jnp.where(take, v, ms[q])
                    ps[q] = jnp.where(take, c, ps[q])
                return ms[0], ms[1], ms[2], ms[3], ps[0], ps[1], ps[2], ps[3]

            m0, m1, m2, m3, p0, p1, p2, p3 = lax.fori_loop(
                0, _C // 8, _cls,
                (neg_inf, neg_inf, neg_inf, neg_inf, zi16, zi16, zi16, zi16))

            t = jnp.logical_or(m1 > m0,
                               jnp.logical_and(m1 == m0, p1 < p0))
            ma = jnp.where(t, m1, m0)
            pa = jnp.where(t, p1, p0)
            t = jnp.logical_or(m3 > m2,
                               jnp.logical_and(m3 == m2, p3 < p2))
            mb = jnp.where(t, m3, m2)
            pb = jnp.where(t, p3, p2)
            t = jnp.logical_or(mb > ma,
                               jnp.logical_and(mb == ma, pb < pa))
            pidx = jnp.where(t, pb, pa)

            l = lab_v[pl.ds(c16 * _CH + g * 16, 16)]
            corr = jnp.where(pidx == l, 1.0, 0.0).astype(jnp.float32)
            addr = lane_base + pidx
            plsc.addupdate_scatter(hpp, [addr], ones)
            plsc.addupdate_scatter(htp, [addr], corr)

    cp_a = pltpu.async_copy(x_hbm.at[pl.ds(base, _CH), :], xa, sem_a)
    cp_b = None
    for c16 in range(_NCH):
        buf = xa if c16 % 2 == 0 else xb
        nxt = xb if c16 % 2 == 0 else xa
        nsem = sem_b if c16 % 2 == 0 else sem_a
        cp_n = None
        if c16 + 1 < _NCH:
            cp_n = pltpu.async_copy(
                x_hbm.at[pl.ds(base + (c16 + 1) * _CH, _CH), :], nxt, nsem)
        if c16 % 2 == 0:
            cp_a.wait()
            cp_b = cp_n
        else:
            cp_b.wait()
            cp_a = cp_n
        _chunk(buf, c16)

    def _lred(k, _):
        app = z16
        atp = z16
        for l2 in range(16):
            app = app + hpp[pl.ds(l2 * _CP + k * 16, 16)]
            atp = atp + htp[pl.ds(l2 * _CP + k * 16, 16)]
        part_pp[pl.ds(k * 16, 16)] = app
        part_tp[pl.ds(k * 16, 16)] = atp
        return 0

    lax.fori_loop(0, _CP // 16, _lred, 0)

    pltpu.sync_copy(part_pp, out_hbm.at[pl.ds(w * _CP, _CP)])
    pltpu.sync_copy(part_tp, out_hbm.at[pl.ds((_NW + w) * _CP, _CP)])


@functools.partial(
    pl.kernel,
    out_type=jax.ShapeDtypeStruct((2 * _NW * _CP,), jnp.float32),
    mesh=plsc.VectorSubcoreMesh(core_axis_name="c", subcore_axis_name="s"),
    compiler_params=pltpu.CompilerParams(needs_layout_passes=False),
    scratch_types=[
        pltpu.VMEM((_CH, _C), jnp.float32),
        pltpu.VMEM((_CH, _C), jnp.float32),
        pltpu.VMEM((_ROWS,), jnp.int32),
        pltpu.VMEM((16 * _CP,), jnp.float32),
        pltpu.VMEM((16 * _CP,), jnp.float32),
        pltpu.VMEM((_CP,), jnp.float32),
        pltpu.VMEM((_CP,), jnp.float32),
        pltpu.SemaphoreType.DMA,
        pltpu.SemaphoreType.DMA,
    ],
)
def _sc_main(x_hbm, lab_hbm, out_hbm, *scratch):
    _sc_body(x_hbm, lab_hbm, out_hbm, *scratch)


def _fin_body(tc_ref, sc_ref, out_ref):
    pp = tc_ref[0:1, :] + jnp.sum(sc_ref[0:_NW, :], axis=0, keepdims=True)
    tp = tc_ref[1:2, :] + jnp.sum(sc_ref[_NW:2 * _NW, :], axis=0,
                                  keepdims=True)
    safe = jnp.where(pp > 0, pp, 1.0)
    prec = jnp.where(pp > 0, tp / safe, 0.0)
    out_ref[...] = jnp.sum(prec, axis=1, keepdims=True) * (1.0 / _C)


def kernel(logits, labels):
    labs3 = labels.reshape(_B // _BM, 1, _BM)
    sc_rows = lax.slice(logits, (_TC_ROWS, 0), (_B, _C))
    sc_parts = _sc_main(sc_rows, labels)
    tc_acc = pl.pallas_call(
        _tc_body,
        grid=(_GRID,),
        in_specs=[
            pl.BlockSpec((_BM, _C), lambda i: (i, 0)),
            pl.BlockSpec((1, 1, _BM), lambda i: (i, 0, 0)),
        ],
        out_specs=pl.BlockSpec((2, _CP), lambda i: (0, 0)),
        out_shape=jax.ShapeDtypeStruct((2, _CP), jnp.float32),
        scratch_shapes=[pltpu.VMEM((2, _CP), jnp.float32)],
    )(logits, labs3)
    out = pl.pallas_call(
        _fin_body,
        out_shape=jax.ShapeDtypeStruct((1, 1), jnp.float32),
    )(tc_acc, sc_parts.reshape(2 * _NW, _CP))
    return out.reshape(())


# speedup vs baseline: 1.1888x; 1.1888x over previous
import functools

import jax
import jax.numpy as jnp
from jax import lax
from jax.experimental import pallas as pl
from jax.experimental.pallas import tpu as pltpu
from jax.experimental.pallas import tpu_sc as plsc

_B = 16384
_C = 1000
_CP = 1024
_BM = 4096
_GRID = _B // _BM
_NS = 16
_PER_TILE = _B // _NS
_PR = _PER_TILE // 128


def _argmax_body(x_ref, out_ref):
    x = x_ref[...]
    m = jnp.max(x, axis=1, keepdims=True)
    idx = lax.broadcasted_iota(jnp.int32, (_BM, _C), 1)
    masked = jnp.where(x == m, idx, _C)
    pred = jnp.min(masked, axis=1)
    out_ref[...] = pred.reshape(_BM // 128, 128)


def _sc_hist_body(pred_hbm, lab_hbm, zi_hbm, out_hbm,
                  pred_v, lab_v, hpp, htp, part_pp, part_tp,
                  zrow, app2, atp2, ov, row0, sh_pp, sh_tp):
    cid = lax.axis_index("c")
    sid = lax.axis_index("s")

    pltpu.sync_copy(zi_hbm, row0)
    pltpu.sync_copy(pred_hbm.at[pl.ds(sid * _PR, _PR), :], pred_v)
    pltpu.sync_copy(lab_hbm.at[pl.ds(sid * _PER_TILE, _PER_TILE)], lab_v)

    z16 = jnp.zeros((16,), jnp.float32)

    def _zero(j, _):
        for u in range(16):
            hpp[pl.ds(j * 256 + u * 16, 16)] = z16
            htp[pl.ds(j * 256 + u * 16, 16)] = z16
        return 0

    lax.fori_loop(0, 16 * _CP // 256, _zero, 0)

    @pl.when(sid == 0)
    def _zero_shared():
        def _zr(j, _):
            for u in range(8):
                zrow[0, pl.ds(j * 128 + u * 16, 16)] = z16
            return 0

        lax.fori_loop(0, _CP // 128, _zr, 0)
        pltpu.sync_copy(zrow, sh_pp)
        pltpu.sync_copy(zrow, sh_tp)

    lane_base = lax.iota(jnp.int32, 16) * _CP
    ones = jnp.ones((16,), jnp.float32)

    def _scat(j, _):
        for u in range(8):
            g = j * 8 + u
            p = pred_v[g // 8, pl.ds((g % 8) * 16, 16)]
            l = lab_v[pl.ds(g * 16, 16)]
            corr = jnp.where(p == l, 1.0, 0.0).astype(jnp.float32)
            addr = lane_base + p
            plsc.addupdate_scatter(hpp, [addr], ones)
            plsc.addupdate_scatter(htp, [addr], corr)
        return 0

    lax.fori_loop(0, _PER_TILE // 16 // 8, _scat, 0)

    def _lred(k, _):
        app = z16
        atp = z16
        for l in range(16):
            app = app + hpp[pl.ds(l * _CP + k * 16, 16)]
            atp = atp + htp[pl.ds(l * _CP + k * 16, 16)]
        part_pp[0, pl.ds(k * 16, 16)] = app
        part_tp[0, pl.ds(k * 16, 16)] = atp
        return 0

    lax.fori_loop(0, _CP // 16, _lred, 0)

    plsc.subcore_barrier()

    pltpu.sync_copy(part_pp, sh_pp.at[row0], add=True)
    pltpu.sync_copy(part_tp, sh_tp.at[row0], add=True)

    plsc.subcore_barrier()

    @pl.when(jnp.logical_and(cid == 0, sid == 0))
    def _fini():
        pltpu.sync_copy(sh_pp, app2)
        pltpu.sync_copy(sh_tp, atp2)

        def _prec(k, psum):
            pp = app2[0, pl.ds(k * 16, 16)]
            tp = atp2[0, pl.ds(k * 16, 16)]
            safe = jnp.where(pp > 0, pp, 1.0)
            return psum + jnp.where(pp > 0, tp / safe, 0.0)

        psum = lax.fori_loop(0, _CP // 16, _prec, z16)
        total = jnp.sum(psum)
        ov[...] = jnp.full((16,), total, jnp.float32) * jnp.float32(1.0 / _C)
        pltpu.sync_copy(ov, out_hbm)


@functools.partial(
    pl.kernel,
    out_type=jax.ShapeDtypeStruct((16,), jnp.float32),
    mesh=plsc.VectorSubcoreMesh(core_axis_name="c", subcore_axis_name="s"),
    compiler_params=pltpu.CompilerParams(needs_layout_passes=False),
    scratch_types=[
        pltpu.VMEM((_PR, 128), jnp.int32),
        pltpu.VMEM((_PER_TILE,), jnp.int32),
        pltpu.VMEM((16 * _CP,), jnp.float32),
        pltpu.VMEM((16 * _CP,), jnp.float32),
        pltpu.VMEM((1, _CP), jnp.float32),
        pltpu.VMEM((1, _CP), jnp.float32),
        pltpu.VMEM((1, _CP), jnp.float32),
        pltpu.VMEM((1, _CP), jnp.float32),
        pltpu.VMEM((1, _CP), jnp.float32),
        pltpu.VMEM((16,), jnp.float32),
        pltpu.VMEM((1,), jnp.int32),
        pltpu.VMEM_SHARED((1, _CP), jnp.float32),
        pltpu.VMEM_SHARED((1, _CP), jnp.float32),
    ],
)
def _sc_hist(pred_hbm, lab_hbm, zi_hbm, out_hbm, *scratch):
    _sc_hist_body(pred_hbm, lab_hbm, zi_hbm, out_hbm, *scratch)


def kernel(logits, labels):
    pred2d = pl.pallas_call(
        _argmax_body,
        grid=(_GRID,),
        in_specs=[pl.BlockSpec((_BM, _C), lambda i: (i, 0))],
        out_specs=pl.BlockSpec((_BM // 128, 128), lambda i: (i, 0)),
        out_shape=jax.ShapeDtypeStruct((_B // 128, 128), jnp.int32),
    )(logits)
    out16 = _sc_hist(pred2d, labels, jnp.zeros((1,), jnp.int32))
    return out16[0].reshape(())


# speedup vs baseline: 1.2005x; 1.0098x over previous
import functools

import jax
import jax.numpy as jnp
from jax import lax
from jax.experimental import pallas as pl
from jax.experimental.pallas import tpu as pltpu
from jax.experimental.pallas import tpu_sc as plsc

_B = 16384
_C = 1000
_CP = 1024
_BM = 2048
_GRID = _B // _BM
_NS = 16
_PER_TILE = _B // _NS
_PR = _PER_TILE // 128


def _argmax_body(x_ref, out_ref):
    x = x_ref[...]
    m = jnp.max(x, axis=1, keepdims=True)
    idx = lax.broadcasted_iota(jnp.int32, (_BM, _C), 1)
    masked = jnp.where(x == m, idx, _C)
    pred = jnp.min(masked, axis=1)
    out_ref[...] = pred.reshape(_BM // 128, 128)


def _sc_hist_body(pred_hbm, lab_hbm, zi_hbm, out_hbm,
                  pred_v, lab_v, hpp, htp, part_pp, part_tp,
                  zrow, app2, atp2, ov, row0, sh_pp, sh_tp):
    cid = lax.axis_index("c")
    sid = lax.axis_index("s")

    pltpu.sync_copy(zi_hbm, row0)
    pltpu.sync_copy(pred_hbm.at[pl.ds(sid * _PR, _PR), :], pred_v)
    pltpu.sync_copy(lab_hbm.at[pl.ds(sid * _PER_TILE, _PER_TILE)], lab_v)

    z16 = jnp.zeros((16,), jnp.float32)

    def _zero(j, _):
        for u in range(16):
            hpp[pl.ds(j * 256 + u * 16, 16)] = z16
            htp[pl.ds(j * 256 + u * 16, 16)] = z16
        return 0

    lax.fori_loop(0, 16 * _CP // 256, _zero, 0)

    @pl.when(sid == 0)
    def _zero_shared():
        def _zr(j, _):
            for u in range(8):
                zrow[0, pl.ds(j * 128 + u * 16, 16)] = z16
            return 0

        lax.fori_loop(0, _CP // 128, _zr, 0)
        pltpu.sync_copy(zrow, sh_pp)
        pltpu.sync_copy(zrow, sh_tp)

    lane_base = lax.iota(jnp.int32, 16) * _CP
    ones = jnp.ones((16,), jnp.float32)

    def _scat(j, _):
        for u in range(8):
            g = j * 8 + u
            p = pred_v[g // 8, pl.ds((g % 8) * 16, 16)]
            l = lab_v[pl.ds(g * 16, 16)]
            corr = jnp.where(p == l, 1.0, 0.0).astype(jnp.float32)
            addr = lane_base + p
            plsc.addupdate_scatter(hpp, [addr], ones)
            plsc.addupdate_scatter(htp, [addr], corr)
        return 0

    lax.fori_loop(0, _PER_TILE // 16 // 8, _scat, 0)

    def _lred(k, _):
        app = z16
        atp = z16
        for l in range(16):
            app = app + hpp[pl.ds(l * _CP + k * 16, 16)]
            atp = atp + htp[pl.ds(l * _CP + k * 16, 16)]
        part_pp[0, pl.ds(k * 16, 16)] = app
        part_tp[0, pl.ds(k * 16, 16)] = atp
        return 0

    lax.fori_loop(0, _CP // 16, _lred, 0)

    plsc.subcore_barrier()

    pltpu.sync_copy(part_pp, sh_pp.at[row0], add=True)
    pltpu.sync_copy(part_tp, sh_tp.at[row0], add=True)

    plsc.subcore_barrier()

    @pl.when(jnp.logical_and(cid == 0, sid == 0))
    def _fini():
        pltpu.sync_copy(sh_pp, app2)
        pltpu.sync_copy(sh_tp, atp2)

        def _prec(k, psum):
            pp = app2[0, pl.ds(k * 16, 16)]
            tp = atp2[0, pl.ds(k * 16, 16)]
            safe = jnp.where(pp > 0, pp, 1.0)
            return psum + jnp.where(pp > 0, tp / safe, 0.0)

        psum = lax.fori_loop(0, _CP // 16, _prec, z16)
        total = jnp.sum(psum)
        ov[...] = jnp.full((16,), total, jnp.float32) * jnp.float32(1.0 / _C)
        pltpu.sync_copy(ov, out_hbm)


@functools.partial(
    pl.kernel,
    out_type=jax.ShapeDtypeStruct((16,), jnp.float32),
    mesh=plsc.VectorSubcoreMesh(core_axis_name="c", subcore_axis_name="s"),
    compiler_params=pltpu.CompilerParams(needs_layout_passes=False),
    scratch_types=[
        pltpu.VMEM((_PR, 128), jnp.int32),
        pltpu.VMEM((_PER_TILE,), jnp.int32),
        pltpu.VMEM((16 * _CP,), jnp.float32),
        pltpu.VMEM((16 * _CP,), jnp.float32),
        pltpu.VMEM((1, _CP), jnp.float32),
        pltpu.VMEM((1, _CP), jnp.float32),
        pltpu.VMEM((1, _CP), jnp.float32),
        pltpu.VMEM((1, _CP), jnp.float32),
        pltpu.VMEM((1, _CP), jnp.float32),
        pltpu.VMEM((16,), jnp.float32),
        pltpu.VMEM((1,), jnp.int32),
        pltpu.VMEM_SHARED((1, _CP), jnp.float32),
        pltpu.VMEM_SHARED((1, _CP), jnp.float32),
    ],
)
def _sc_hist(pred_hbm, lab_hbm, zi_hbm, out_hbm, *scratch):
    _sc_hist_body(pred_hbm, lab_hbm, zi_hbm, out_hbm, *scratch)


def kernel(logits, labels):
    pred2d = pl.pallas_call(
        _argmax_body,
        grid=(_GRID,),
        in_specs=[pl.BlockSpec((_BM, _C), lambda i: (i, 0))],
        out_specs=pl.BlockSpec((_BM // 128, 128), lambda i: (i, 0)),
        out_shape=jax.ShapeDtypeStruct((_B // 128, 128), jnp.int32),
    )(logits)
    out16 = _sc_hist(pred2d, labels, jnp.zeros((1,), jnp.int32))
    return out16[0].reshape(())
